# Initial kernel scaffold; baseline (speedup 1.0000x reference)
import jax
import jax.numpy as jnp
from jax.experimental import pallas as pl


def kernel(centers, contexts_negatives, embed_v_weight, embed_u_weight):
    raise NotImplementedError("write your pallas kernel here")


# speedup vs baseline: 20.0788x; 20.0788x over previous
import functools

import jax
import jax.numpy as jnp
from jax import lax
from jax.experimental import pallas as pl
from jax.experimental.pallas import tpu as pltpu
from jax.experimental.pallas import tpu_sc as plsc

VOCAB_PAD = 1024
L_PAD = 208
CHUNK = 16


def _gram_body(v_ref, u_ref, g_ref):
    g_ref[...] = lax.dot_general(
        v_ref[...], u_ref[...],
        (((1,), (1,)), ((), ())),
        preferred_element_type=jnp.float32,
    )


@functools.lru_cache(maxsize=None)
def _make_sc_gather(num_chunks: int):
    info = plsc.get_sparse_core_info()
    nc, ns = info.num_cores, info.num_subcores
    nw = nc * ns
    chunks_per_tile = num_chunks // nw
    mesh = plsc.VectorSubcoreMesh(core_axis_name="c", subcore_axis_name="s")

    @functools.partial(
        pl.kernel,
        mesh=mesh,
        compiler_params=pltpu.CompilerParams(
            use_tc_tiling_on_sc=False, needs_layout_passes=False),
        out_type=jax.ShapeDtypeStruct((num_chunks, L_PAD, CHUNK), jnp.float32),
        scratch_types=[
            pltpu.VMEM((chunks_per_tile, CHUNK), jnp.int32),
            pltpu.VMEM((L_PAD, CHUNK), jnp.int32),
            pltpu.VMEM((CHUNK, VOCAB_PAD), jnp.float32),
            pltpu.VMEM((L_PAD, CHUNK), jnp.float32),
            pltpu.SemaphoreType.DMA,
        ],
    )
    def sc_gather(g_hbm, ctx_hbm, cen_hbm, out_hbm,
                  cen_v, ctx_v, rows_v, out_v, sem):
        wid = lax.axis_index("s") * nc + lax.axis_index("c")
        base = wid * chunks_per_tile
        pltpu.sync_copy(cen_hbm.at[pl.ds(base, chunks_per_tile)], cen_v)
        row_ids = lax.iota(jnp.int32, 16)

        def chunk_body(k, carry):
            c = base + k
            pltpu.sync_copy(ctx_hbm.at[c], ctx_v)
            pltpu.async_copy(g_hbm.at[cen_v.at[k]], rows_v, sem).wait()

            def j_body(j, carry2):
                idx = ctx_v[j]
                out_v[j] = plsc.load_gather(rows_v, [row_ids, idx])
                return carry2

            lax.fori_loop(0, L_PAD, j_body, 0, unroll=4)
            pltpu.sync_copy(out_v, out_hbm.at[c])
            return carry

        lax.fori_loop(0, chunks_per_tile, chunk_body, 0)

    return sc_gather


def kernel(centers, contexts_negatives, embed_v_weight, embed_u_weight):
    B = centers.shape[0]
    L = contexts_negatives.shape[1]
    V = embed_v_weight.shape[0]
    num_chunks = B // CHUNK

    v_pad = jnp.pad(embed_v_weight, ((0, VOCAB_PAD - V), (0, 0)))
    u_pad = jnp.pad(embed_u_weight, ((0, VOCAB_PAD - V), (0, 0)))
    g = pl.pallas_call(
        _gram_body,
        out_shape=jax.ShapeDtypeStruct((VOCAB_PAD, VOCAB_PAD), jnp.float32),
    )(v_pad, u_pad)

    cen = centers.astype(jnp.int32).reshape(num_chunks, CHUNK)
    ctx = contexts_negatives.astype(jnp.int32)
    ctx_t = jnp.pad(ctx, ((0, 0), (0, L_PAD - L)))
    ctx_t = ctx_t.reshape(num_chunks, CHUNK, L_PAD).transpose(0, 2, 1)

    out_t = _make_sc_gather(num_chunks)(g, ctx_t, cen)
    out = out_t.transpose(0, 2, 1).reshape(B, L_PAD)[:, :L]
    return out.reshape(B, 1, L)


# speedup vs baseline: 36.5102x; 1.8183x over previous
import functools

import jax
import jax.numpy as jnp
import numpy as np
from jax import lax
from jax.experimental import pallas as pl
from jax.experimental.pallas import tpu as pltpu
from jax.experimental.pallas import tpu_sc as plsc

VOCAB_PAD = 1024
CHUNK = 16


def _make_gram_body(V):
    pad = VOCAB_PAD - V

    def gram_body(v_ref, u_ref, g_ref):
        z = jnp.zeros((pad, v_ref.shape[1]), jnp.float32)
        vp = jnp.concatenate([v_ref[...], z], axis=0)
        up = jnp.concatenate([u_ref[...], z], axis=0)
        g_ref[...] = lax.dot_general(
            vp, up, (((1,), (1,)), ((), ())),
            preferred_element_type=jnp.float32,
        )

    return gram_body


@functools.lru_cache(maxsize=None)
def _make_sc_gather(num_chunks: int, L: int):
    info = plsc.get_sparse_core_info()
    nc, ns = info.num_cores, info.num_subcores
    nw = nc * ns
    chunks_per_tile = num_chunks // nw
    flat = CHUNK * L
    groups = flat // 16
    mesh = plsc.VectorSubcoreMesh(core_axis_name="c", subcore_axis_name="s")

    @functools.partial(
        pl.kernel,
        mesh=mesh,
        compiler_params=pltpu.CompilerParams(
            use_tc_tiling_on_sc=False, needs_layout_passes=False),
        out_type=jax.ShapeDtypeStruct((num_chunks * flat,), jnp.float32),
        scratch_types=[
            pltpu.VMEM((chunks_per_tile, CHUNK), jnp.int32),
            pltpu.VMEM((flat,), jnp.int32),
            pltpu.VMEM((flat,), jnp.int32),
            pltpu.VMEM((CHUNK, VOCAB_PAD), jnp.float32),
            pltpu.VMEM((flat,), jnp.float32),
            pltpu.SemaphoreType.DMA,
        ],
    )
    def sc_gather(g_hbm, ctx_hbm, cen_hbm, rid_hbm, out_hbm,
                  cen_v, rid_v, ctx_v, rows_v, out_v, sem):
        wid = lax.axis_index("s") * nc + lax.axis_index("c")
        base = wid * chunks_per_tile
        pltpu.sync_copy(cen_hbm.at[pl.ds(base, chunks_per_tile)], cen_v)
        pltpu.sync_copy(rid_hbm, rid_v)

        def chunk_body(k, carry):
            c = base + k
            pltpu.sync_copy(ctx_hbm.at[pl.ds(c * flat, flat)], ctx_v)
            pltpu.async_copy(g_hbm.at[cen_v.at[k]], rows_v, sem).wait()
            for t in range(groups):
                sl = pl.ds(t * 16, 16)
                out_v[sl] = plsc.load_gather(rows_v, [rid_v[sl], ctx_v[sl]])
            pltpu.sync_copy(out_v, out_hbm.at[pl.ds(c * flat, flat)])
            return carry

        lax.fori_loop(0, chunks_per_tile, chunk_body, 0)

    return sc_gather


def kernel(centers, contexts_negatives, embed_v_weight, embed_u_weight):
    B = centers.shape[0]
    L = contexts_negatives.shape[1]
    V = embed_v_weight.shape[0]
    num_chunks = B // CHUNK

    g = pl.pallas_call(
        _make_gram_body(V),
        out_shape=jax.ShapeDtypeStruct((VOCAB_PAD, VOCAB_PAD), jnp.float32),
    )(embed_v_weight, embed_u_weight)

    cen = centers.astype(jnp.int32).reshape(num_chunks, CHUNK)
    ctx = contexts_negatives.astype(jnp.int32).reshape(-1)
    rid = jnp.asarray(np.arange(CHUNK * L, dtype=np.int32) // L)

    out = _make_sc_gather(num_chunks, L)(g, ctx, cen, rid)
    return out.reshape(B, 1, L)


# speedup vs baseline: 39.1320x; 1.0718x over previous
import functools

import jax
import jax.numpy as jnp
import numpy as np
from jax import lax
from jax.experimental import pallas as pl
from jax.experimental.pallas import tpu as pltpu
from jax.experimental.pallas import tpu_sc as plsc

VOCAB_PAD = 1024
CHUNK = 16


def _make_gram_body(V):
    pad = VOCAB_PAD - V

    def gram_body(v_ref, u_ref, g_ref):
        z = jnp.zeros((pad, v_ref.shape[1]), jnp.float32)
        vp = jnp.concatenate([v_ref[...], z], axis=0)
        up = jnp.concatenate([u_ref[...], z], axis=0)
        g_ref[...] = lax.dot_general(
            vp, up, (((1,), (1,)), ((), ())),
            preferred_element_type=jnp.float32,
        )

    return gram_body


@functools.lru_cache(maxsize=None)
def _make_sc_gather(B: int, L: int):
    info = plsc.get_sparse_core_info()
    nc, ns = info.num_cores, info.num_subcores
    nw = nc * ns
    num_chunks = B // CHUNK
    chunks_per_tile = num_chunks // nw
    rows_per_tile = chunks_per_tile * CHUNK
    offs = list(range(0, L - 15, 16))
    if offs[-1] + 16 < L:
        offs.append(L - 16)
    mesh = plsc.VectorSubcoreMesh(core_axis_name="c", subcore_axis_name="s")

    @functools.partial(
        pl.kernel,
        mesh=mesh,
        compiler_params=pltpu.CompilerParams(
            use_tc_tiling_on_sc=False, needs_layout_passes=False),
        out_type=jax.ShapeDtypeStruct((B, L), jnp.float32),
        scratch_types=[
            pltpu.VMEM((chunks_per_tile, CHUNK), jnp.int32),
            pltpu.VMEM((CHUNK, CHUNK), jnp.int32),
            pltpu.VMEM((rows_per_tile, L), jnp.int32),
            pltpu.VMEM((CHUNK, VOCAB_PAD), jnp.float32),
            pltpu.VMEM((CHUNK, VOCAB_PAD), jnp.float32),
            pltpu.VMEM((rows_per_tile, L), jnp.float32),
            pltpu.SemaphoreType.DMA,
            pltpu.SemaphoreType.DMA,
            pltpu.SemaphoreType.DMA,
        ],
    )
    def sc_gather(g_hbm, ctx_hbm, cen_hbm, rid_hbm, out_hbm,
                  cen_v, rid_v, ctx_v, rows0, rows1, out_v,
                  ctx_sem, rsem0, rsem1):
        wid = lax.axis_index("s") * nc + lax.axis_index("c")
        base_chunk = wid * chunks_per_tile
        base_row = base_chunk * CHUNK
        pltpu.sync_copy(cen_hbm.at[pl.ds(base_chunk, chunks_per_tile)], cen_v)
        pltpu.sync_copy(rid_hbm, rid_v)
        ctx_h = pltpu.async_copy(
            ctx_hbm.at[pl.ds(base_row, rows_per_tile)], ctx_v, ctx_sem)

        rows = [rows0, rows1]
        rsems = [rsem0, rsem1]
        row_h = [None, None]
        row_h[0] = pltpu.async_copy(g_hbm.at[cen_v.at[0]], rows0, rsem0)

        rowvecs = [rid_v[b] for b in range(CHUNK)]
        ctx_h.wait()

        for k in range(chunks_per_tile):
            if k + 1 < chunks_per_tile:
                nb = (k + 1) % 2
                row_h[nb] = pltpu.async_copy(
                    g_hbm.at[cen_v.at[k + 1]], rows[nb], rsems[nb])
            row_h[k % 2].wait()
            rbuf = rows[k % 2]
            for b in range(CHUNK):
                r = k * CHUNK + b
                for o in offs:
                    sl = pl.ds(o, 16)
                    out_v[r, sl] = plsc.load_gather(
                        rbuf, [rowvecs[b], ctx_v[r, sl]])

        pltpu.sync_copy(out_v, out_hbm.at[pl.ds(base_row, rows_per_tile)])

    return sc_gather


def kernel(centers, contexts_negatives, embed_v_weight, embed_u_weight):
    B = centers.shape[0]
    L = contexts_negatives.shape[1]
    V = embed_v_weight.shape[0]

    g = pl.pallas_call(
        _make_gram_body(V),
        out_shape=jax.ShapeDtypeStruct((VOCAB_PAD, VOCAB_PAD), jnp.float32),
    )(embed_v_weight, embed_u_weight)

    cen = centers.astype(jnp.int32).reshape(B // CHUNK, CHUNK)
    ctx = contexts_negatives.astype(jnp.int32)
    rid = jnp.asarray(
        np.repeat(np.arange(CHUNK, dtype=np.int32), CHUNK).reshape(
            CHUNK, CHUNK))

    out = _make_sc_gather(B, L)(g, ctx, cen, rid)
    return out.reshape(B, 1, L)


# speedup vs baseline: 42.8955x; 1.0962x over previous
import functools

import jax
import jax.numpy as jnp
import numpy as np
from jax import lax
from jax.experimental import pallas as pl
from jax.experimental.pallas import tpu as pltpu
from jax.experimental.pallas import tpu_sc as plsc

VOCAB_PAD = 1024
CHUNK = 16
LANE = 128


def _gram_body(v_ref, u_ref, g_ref):
    gg = lax.dot_general(
        v_ref[...], u_ref[...], (((1,), (1,)), ((), ())),
        preferred_element_type=jnp.float32,
    )
    g_ref[...] = gg.reshape(g_ref.shape)


@functools.lru_cache(maxsize=None)
def _make_sc_gather(B: int, L: int):
    info = plsc.get_sparse_core_info()
    nc, ns = info.num_cores, info.num_subcores
    nw = nc * ns
    num_chunks = B // CHUNK
    chunks_per_tile = num_chunks // nw
    rows_per_tile = chunks_per_tile * CHUNK
    offs = list(range(0, L - 15, 16))
    if offs[-1] + 16 < L:
        offs.append(L - 16)
    mesh = plsc.VectorSubcoreMesh(core_axis_name="c", subcore_axis_name="s")

    @functools.partial(
        pl.kernel,
        mesh=mesh,
        compiler_params=pltpu.CompilerParams(
            use_tc_tiling_on_sc=False, needs_layout_passes=False),
        out_type=jax.ShapeDtypeStruct((B, L), jnp.float32),
        scratch_types=[
            pltpu.VMEM((chunks_per_tile, CHUNK), jnp.int32),
            pltpu.VMEM((CHUNK, CHUNK), jnp.int32),
            pltpu.VMEM((rows_per_tile, L), jnp.int32),
            pltpu.VMEM((CHUNK, VOCAB_PAD // LANE, LANE), jnp.float32),
            pltpu.VMEM((CHUNK, VOCAB_PAD // LANE, LANE), jnp.float32),
            pltpu.VMEM((rows_per_tile, L), jnp.float32),
            pltpu.SemaphoreType.DMA,
            pltpu.SemaphoreType.DMA,
            pltpu.SemaphoreType.DMA,
        ],
    )
    def sc_gather(g_hbm, ctx_hbm, cen_hbm, rid_hbm, out_hbm,
                  cen_v, rid_v, ctx_v, rows0, rows1, out_v,
                  ctx_sem, rsem0, rsem1):
        wid = lax.axis_index("s") * nc + lax.axis_index("c")
        base_chunk = wid * chunks_per_tile
        base_row = base_chunk * CHUNK
        pltpu.sync_copy(cen_hbm.at[pl.ds(base_chunk, chunks_per_tile)], cen_v)
        pltpu.sync_copy(rid_hbm, rid_v)
        ctx_h = pltpu.async_copy(
            ctx_hbm.at[pl.ds(base_row, rows_per_tile)], ctx_v, ctx_sem)

        rows = [rows0, rows1]
        rsems = [rsem0, rsem1]
        row_h = [None, None]
        row_h[0] = pltpu.async_copy(g_hbm.at[cen_v.at[0]], rows0, rsem0)

        rowvecs = [rid_v[b] for b in range(CHUNK)]
        ctx_h.wait()

        for k in range(chunks_per_tile):
            if k + 1 < chunks_per_tile:
                nb = (k + 1) % 2
                row_h[nb] = pltpu.async_copy(
                    g_hbm.at[cen_v.at[k + 1]], rows[nb], rsems[nb])
            row_h[k % 2].wait()
            rbuf = rows[k % 2]
            for b in range(CHUNK):
                r = k * CHUNK + b
                for o in offs:
                    sl = pl.ds(o, 16)
                    idx = ctx_v[r, sl]
                    out_v[r, sl] = plsc.load_gather(
                        rbuf, [rowvecs[b], idx >> 7, idx & 127])

        pltpu.sync_copy(out_v, out_hbm.at[pl.ds(base_row, rows_per_tile)])

    return sc_gather


def kernel(centers, contexts_negatives, embed_v_weight, embed_u_weight):
    B = centers.shape[0]
    L = contexts_negatives.shape[1]
    V = embed_v_weight.shape[0]
    nblk = VOCAB_PAD // LANE

    v_pad = jnp.pad(embed_v_weight, ((0, VOCAB_PAD - V), (0, 0)))
    u_pad = jnp.pad(embed_u_weight, ((0, VOCAB_PAD - V), (0, 0)))
    g = pl.pallas_call(
        _gram_body,
        out_shape=jax.ShapeDtypeStruct((VOCAB_PAD, nblk, LANE), jnp.float32),
    )(v_pad, u_pad)

    cen = centers.astype(jnp.int32).reshape(B // CHUNK, CHUNK)
    ctx = contexts_negatives.astype(jnp.int32)
    rid = jnp.asarray(
        np.repeat(np.arange(CHUNK, dtype=np.int32), CHUNK).reshape(
            CHUNK, CHUNK))

    out = _make_sc_gather(B, L)(g, ctx, cen, rid)
    return out.reshape(B, 1, L)


# speedup vs baseline: 49.8299x; 1.1617x over previous
import functools

import jax
import jax.numpy as jnp
from jax import lax
from jax.experimental import pallas as pl
from jax.experimental.pallas import tpu as pltpu
from jax.experimental.pallas import tpu_sc as plsc

VOCAB_PAD = 1024
CHUNK = 16
LANE = 128


def _gram_body(v_ref, u_ref, g_ref):
    gg = lax.dot_general(
        v_ref[...], u_ref[...], (((1,), (1,)), ((), ())),
        preferred_element_type=jnp.float32,
    )
    g_ref[...] = gg.reshape(g_ref.shape)


@functools.lru_cache(maxsize=None)
def _make_sc_gather(B: int, L: int):
    info = plsc.get_sparse_core_info()
    nc, ns = info.num_cores, info.num_subcores
    nw = nc * ns
    num_chunks = B // CHUNK
    chunks_per_tile = num_chunks // nw
    rows_per_tile = chunks_per_tile * CHUNK
    offs = list(range(0, L - 15, 16))
    if offs[-1] + 16 < L:
        offs.append(L - 16)
    mesh = plsc.VectorSubcoreMesh(core_axis_name="c", subcore_axis_name="s")

    @functools.partial(
        pl.kernel,
        mesh=mesh,
        compiler_params=pltpu.CompilerParams(
            use_tc_tiling_on_sc=True, needs_layout_passes=False),
        out_type=jax.ShapeDtypeStruct((B, L), jnp.float32),
        scratch_types=[
            pltpu.VMEM((rows_per_tile,), jnp.int32),
            pltpu.VMEM((rows_per_tile, L), jnp.int32),
            pltpu.VMEM((CHUNK, VOCAB_PAD // LANE, LANE), jnp.float32),
            pltpu.VMEM((CHUNK, VOCAB_PAD // LANE, LANE), jnp.float32),
            pltpu.VMEM((rows_per_tile, L), jnp.float32),
            pltpu.SemaphoreType.DMA,
            pltpu.SemaphoreType.DMA,
            pltpu.SemaphoreType.DMA,
        ],
    )
    def sc_gather(g_hbm, ctx_hbm, cen_hbm, out_hbm,
                  cen_v, ctx_v, rows0, rows1, out_v,
                  ctx_sem, rsem0, rsem1):
        wid = lax.axis_index("s") * nc + lax.axis_index("c")
        base_row = wid * rows_per_tile
        pltpu.sync_copy(cen_hbm.at[pl.ds(base_row, rows_per_tile)], cen_v)
        ctx_h = pltpu.async_copy(
            ctx_hbm.at[pl.ds(base_row, rows_per_tile)], ctx_v, ctx_sem)

        rows = [rows0, rows1]
        rsems = [rsem0, rsem1]
        row_h = [None, None]
        row_h[0] = pltpu.async_copy(
            g_hbm.at[cen_v.at[pl.ds(0, CHUNK)]], rows0, rsem0)

        zero16 = lax.iota(jnp.int32, 16) * 0
        rowvecs = [zero16 + b for b in range(CHUNK)]
        ctx_h.wait()

        for k in range(chunks_per_tile):
            if k + 1 < chunks_per_tile:
                nb = (k + 1) % 2
                row_h[nb] = pltpu.async_copy(
                    g_hbm.at[cen_v.at[pl.ds((k + 1) * CHUNK, CHUNK)]],
                    rows[nb], rsems[nb])
            row_h[k % 2].wait()
            rbuf = rows[k % 2]
            for b in range(CHUNK):
                r = k * CHUNK + b
                for o in offs:
                    sl = pl.ds(o, 16)
                    idx = ctx_v[r, sl]
                    out_v[r, sl] = plsc.load_gather(
                        rbuf, [rowvecs[b], idx >> 7, idx & 127])

        pltpu.sync_copy(out_v, out_hbm.at[pl.ds(base_row, rows_per_tile)])

    return sc_gather


def kernel(centers, contexts_negatives, embed_v_weight, embed_u_weight):
    B = centers.shape[0]
    L = contexts_negatives.shape[1]
    V = embed_v_weight.shape[0]
    nblk = VOCAB_PAD // LANE

    v_pad = jnp.pad(embed_v_weight, ((0, VOCAB_PAD - V), (0, 0)))
    u_pad = jnp.pad(embed_u_weight, ((0, VOCAB_PAD - V), (0, 0)))
    g = pl.pallas_call(
        _gram_body,
        out_shape=jax.ShapeDtypeStruct((VOCAB_PAD, nblk, LANE), jnp.float32),
    )(v_pad, u_pad)

    cen = centers.astype(jnp.int32).reshape(B)
    ctx = contexts_negatives.astype(jnp.int32)

    out = _make_sc_gather(B, L)(g, ctx, cen)
    return out.reshape(B, 1, L)


# speedup vs baseline: 50.6729x; 1.0169x over previous
import functools

import jax
import jax.numpy as jnp
from jax import lax
from jax.experimental import pallas as pl
from jax.experimental.pallas import tpu as pltpu
from jax.experimental.pallas import tpu_sc as plsc

VOCAB_PAD = 1024
CHUNK = 16
LANE = 128


def _gram_body(v_ref, u_ref, g_ref):
    gg = lax.dot_general(
        v_ref[...], u_ref[...], (((1,), (1,)), ((), ())),
        preferred_element_type=jnp.float32,
    )
    g_ref[...] = gg.reshape(g_ref.shape)


@functools.lru_cache(maxsize=None)
def _make_sc_gather(B: int, L: int):
    info = plsc.get_sparse_core_info()
    nc, ns = info.num_cores, info.num_subcores
    nw = nc * ns
    num_chunks = B // CHUNK
    chunks_per_tile = num_chunks // nw
    rows_per_tile = chunks_per_tile * CHUNK
    npairs = chunks_per_tile // 2
    offs = list(range(0, L - 15, 16))
    if offs[-1] + 16 < L:
        offs.append(L - 16)
    mesh = plsc.VectorSubcoreMesh(core_axis_name="c", subcore_axis_name="s")

    @functools.partial(
        pl.kernel,
        mesh=mesh,
        compiler_params=pltpu.CompilerParams(
            use_tc_tiling_on_sc=True, needs_layout_passes=False),
        out_type=jax.ShapeDtypeStruct((B, 1, L), jnp.float32),
        scratch_types=[
            pltpu.VMEM((rows_per_tile,), jnp.int32),
            pltpu.VMEM((rows_per_tile, L), jnp.int32),
            pltpu.VMEM((CHUNK, VOCAB_PAD // LANE, LANE), jnp.float32),
            pltpu.VMEM((CHUNK, VOCAB_PAD // LANE, LANE), jnp.float32),
            pltpu.VMEM((rows_per_tile, L), jnp.float32),
            pltpu.SemaphoreType.DMA,
            pltpu.SemaphoreType.DMA,
            pltpu.SemaphoreType.DMA,
        ],
    )
    def sc_gather(g_hbm, ctx_hbm, cen_hbm, out_hbm,
                  cen_v, ctx_v, rows0, rows1, out_v,
                  ctx_sem, rsem0, rsem1):
        wid = lax.axis_index("s") * nc + lax.axis_index("c")
        base_row = wid * rows_per_tile
        pltpu.sync_copy(cen_hbm.at[pl.ds(base_row, rows_per_tile)], cen_v)
        ctx_h = pltpu.async_copy(
            ctx_hbm.at[pl.ds(base_row, rows_per_tile)], ctx_v, ctx_sem)

        zero16 = lax.iota(jnp.int32, 16) * 0
        rowvecs = [zero16 + b for b in range(CHUNK)]

        pltpu.async_copy(g_hbm.at[cen_v.at[pl.ds(0, CHUNK)]], rows0, rsem0)
        ctx_h.wait()

        def compute_chunk(row0, rbuf):
            for b in range(CHUNK):
                r = row0 + b
                for o in offs:
                    sl = pl.ds(o, 16)
                    idx = ctx_v[r, sl]
                    out_v[r, sl] = plsc.load_gather(
                        rbuf, [rowvecs[b], idx >> 7, idx & 127])

        def pair_body(j, carry):
            r0 = j * (2 * CHUNK)
            pltpu.async_copy(
                g_hbm.at[cen_v.at[pl.ds(r0 + CHUNK, CHUNK)]], rows1, rsem1)
            pltpu.make_async_copy(
                g_hbm.at[cen_v.at[pl.ds(0, CHUNK)]], rows0, rsem0).wait()
            compute_chunk(r0, rows0)
            nxt = jnp.minimum(r0 + 2 * CHUNK, rows_per_tile - CHUNK)
            pltpu.async_copy(
                g_hbm.at[cen_v.at[pl.ds(nxt, CHUNK)]], rows0, rsem0)
            pltpu.make_async_copy(
                g_hbm.at[cen_v.at[pl.ds(0, CHUNK)]], rows1, rsem1).wait()
            compute_chunk(r0 + CHUNK, rows1)
            return carry

        lax.fori_loop(0, npairs, pair_body, 0)
        pltpu.make_async_copy(
            g_hbm.at[cen_v.at[pl.ds(0, CHUNK)]], rows0, rsem0).wait()

        pltpu.sync_copy(out_v, out_hbm.at[pl.ds(base_row, rows_per_tile), 0])

    return sc_gather


def kernel(centers, contexts_negatives, embed_v_weight, embed_u_weight):
    B = centers.shape[0]
    L = contexts_negatives.shape[1]
    V = embed_v_weight.shape[0]
    nblk = VOCAB_PAD // LANE

    v_pad = jnp.pad(embed_v_weight, ((0, VOCAB_PAD - V), (0, 0)))
    u_pad = jnp.pad(embed_u_weight, ((0, VOCAB_PAD - V), (0, 0)))
    g = pl.pallas_call(
        _gram_body,
        out_shape=jax.ShapeDtypeStruct((VOCAB_PAD, nblk, LANE), jnp.float32),
    )(v_pad, u_pad)

    cen = centers.astype(jnp.int32).reshape(B)
    ctx = contexts_negatives.astype(jnp.int32)

    return _make_sc_gather(B, L)(g, ctx, cen)


# speedup vs baseline: 55.4954x; 1.0952x over previous
import functools

import jax
import jax.numpy as jnp
from jax import lax
from jax.experimental import pallas as pl
from jax.experimental.pallas import tpu as pltpu
from jax.experimental.pallas import tpu_sc as plsc

VOCAB_PAD = 1024
CHUNK = 16
LANE = 128


def _gram_body(v_ref, u_ref, g_ref):
    gg = lax.dot_general(
        v_ref[...], u_ref[...], (((1,), (1,)), ((), ())),
        preferred_element_type=jnp.float32,
    )
    g_ref[...] = gg.reshape(g_ref.shape)


@functools.lru_cache(maxsize=None)
def _make_sc_gather(B: int, L: int):
    info = plsc.get_sparse_core_info()
    nc, ns = info.num_cores, info.num_subcores
    nw = nc * ns
    num_chunks = B // CHUNK
    chunks_per_tile = num_chunks // nw
    rows_per_tile = chunks_per_tile * CHUNK
    npairs = chunks_per_tile // 2
    offs = list(range(0, L - 15, 16))
    if offs[-1] + 16 < L:
        offs.append(L - 16)
    mesh = plsc.VectorSubcoreMesh(core_axis_name="c", subcore_axis_name="s")

    @functools.partial(
        pl.kernel,
        mesh=mesh,
        compiler_params=pltpu.CompilerParams(
            use_tc_tiling_on_sc=True, needs_layout_passes=False),
        out_type=jax.ShapeDtypeStruct((B, L), jnp.float32),
        scratch_types=[
            pltpu.VMEM((rows_per_tile,), jnp.int32),
            pltpu.VMEM((rows_per_tile, L), jnp.int32),
            pltpu.VMEM((CHUNK, VOCAB_PAD // LANE, LANE), jnp.float32),
            pltpu.VMEM((CHUNK, VOCAB_PAD // LANE, LANE), jnp.float32),
            pltpu.VMEM((rows_per_tile, L), jnp.float32),
            pltpu.SemaphoreType.DMA,
            pltpu.SemaphoreType.DMA,
            pltpu.SemaphoreType.DMA,
        ],
    )
    def sc_gather(g_hbm, ctx_hbm, cen_hbm, out_hbm,
                  cen_v, ctx_v, rows0, rows1, out_v,
                  ctx_sem, rsem0, rsem1):
        wid = lax.axis_index("s") * nc + lax.axis_index("c")
        base_row = wid * rows_per_tile
        pltpu.sync_copy(cen_hbm.at[pl.ds(base_row, rows_per_tile)], cen_v)
        ctx_h = pltpu.async_copy(
            ctx_hbm.at[pl.ds(base_row, rows_per_tile)], ctx_v, ctx_sem)

        zero16 = lax.iota(jnp.int32, 16) * 0
        rowvecs = [zero16 + b for b in range(CHUNK)]

        pltpu.async_copy(g_hbm.at[cen_v.at[pl.ds(0, CHUNK)]], rows0, rsem0)
        ctx_h.wait()

        def compute_chunk(row0, rbuf):
            for b in range(CHUNK):
                r = row0 + b
                for o in offs:
                    sl = pl.ds(o, 16)
                    idx = ctx_v[r, sl]
                    out_v[r, sl] = plsc.load_gather(
                        rbuf, [rowvecs[b], idx >> 7, idx & 127])

        def pair_body(j, carry):
            r0 = j * (2 * CHUNK)
            pltpu.async_copy(
                g_hbm.at[cen_v.at[pl.ds(r0 + CHUNK, CHUNK)]], rows1, rsem1)
            pltpu.make_async_copy(
                g_hbm.at[cen_v.at[pl.ds(0, CHUNK)]], rows0, rsem0).wait()
            compute_chunk(r0, rows0)
            nxt = jnp.minimum(r0 + 2 * CHUNK, rows_per_tile - CHUNK)
            pltpu.async_copy(
                g_hbm.at[cen_v.at[pl.ds(nxt, CHUNK)]], rows0, rsem0)
            pltpu.make_async_copy(
                g_hbm.at[cen_v.at[pl.ds(0, CHUNK)]], rows1, rsem1).wait()
            compute_chunk(r0 + CHUNK, rows1)
            return carry

        lax.fori_loop(0, npairs, pair_body, 0)
        pltpu.make_async_copy(
            g_hbm.at[cen_v.at[pl.ds(0, CHUNK)]], rows0, rsem0).wait()

        pltpu.sync_copy(out_v, out_hbm.at[pl.ds(base_row, rows_per_tile)])

    return sc_gather


def kernel(centers, contexts_negatives, embed_v_weight, embed_u_weight):
    B = centers.shape[0]
    L = contexts_negatives.shape[1]
    V = embed_v_weight.shape[0]
    nblk = VOCAB_PAD // LANE

    v_pad = jnp.pad(embed_v_weight, ((0, VOCAB_PAD - V), (0, 0)))
    u_pad = jnp.pad(embed_u_weight, ((0, VOCAB_PAD - V), (0, 0)))
    g = pl.pallas_call(
        _gram_body,
        out_shape=jax.ShapeDtypeStruct((VOCAB_PAD, nblk, LANE), jnp.float32),
    )(v_pad, u_pad)

    cen = centers.astype(jnp.int32).reshape(B)
    ctx = contexts_negatives.astype(jnp.int32)

    out = _make_sc_gather(B, L)(g, ctx, cen)
    return out.reshape(B, 1, L)
